# baseline (device time: 282730 ns/iter reference)
import math

import jax
import jax.numpy as jnp
from jax import lax
from jax.experimental import pallas as pl
from jax.experimental.pallas import tpu as pltpu

N_DEV = 4
SQ = 2048
D = 1024
HQ = 8
DH = 128
QB = 512
SCALE = 0.08838834764831843 * 1.4426950408889634


def _rope_tables(off):
    ri = lax.broadcasted_iota(jnp.int32, (SQ, DH), 0).astype(jnp.float32)
    ci = lax.broadcasted_iota(jnp.int32, (SQ, DH), 1)
    f = (ci // 2).astype(jnp.float32)
    inv = jnp.exp(f * (-math.log(10000.0) / (DH // 2)))
    ang = (off.astype(jnp.float32) + ri) * inv
    return jnp.cos(ang), jnp.sin(ang)


def _rot_mat():
    i = lax.broadcasted_iota(jnp.int32, (DH, DH), 0)
    j = lax.broadcasted_iota(jnp.int32, (DH, DH), 1)
    plus = (j == i + 1) & (i % 2 == 0)
    minus = (j == i - 1) & (i % 2 == 1)
    return plus.astype(jnp.float32) - minus.astype(jnp.float32)


_HG = 4
_WG = _HG * DH


def _qkv_body(x_ref, wq_ref, wk_ref, wv_ref, q_ref, k_ref, v_ref,
              cos_ref, sin_ref):
    @pl.when(pl.program_id(0) == 0)
    def _():
        off = lax.axis_index("i") * SQ
        c, s = _rope_tables(off)
        cos_ref[...] = c
        sin_ref[...] = s

    cos = cos_ref[...]
    sin = sin_ref[...]
    rot = _rot_mat().astype(jnp.bfloat16)
    x = x_ref[...].astype(jnp.bfloat16)

    tq = jnp.dot(x, wq_ref[...].astype(jnp.bfloat16),
                 preferred_element_type=jnp.float32)
    tk = jnp.dot(x, wk_ref[...].astype(jnp.bfloat16),
                 preferred_element_type=jnp.float32)
    tv = jnp.dot(x, wv_ref[...].astype(jnp.bfloat16),
                 preferred_element_type=jnp.float32)
    for hh in range(_HG):
        t = tq[:, hh * DH:(hh + 1) * DH]
        tr = jnp.dot(t.astype(jnp.bfloat16), rot,
                     preferred_element_type=jnp.float32)
        q_ref[hh] = ((t * cos + tr * sin) * SCALE).astype(jnp.bfloat16)
        t = tk[:, hh * DH:(hh + 1) * DH]
        tr = jnp.dot(t.astype(jnp.bfloat16), rot,
                     preferred_element_type=jnp.float32)
        k_ref[hh] = (t * cos + tr * sin).astype(jnp.bfloat16)
        v_ref[hh] = tv[:, hh * DH:(hh + 1) * DH].astype(jnp.bfloat16)


def _qkv(x, Wq, Wk, Wv):
    return pl.pallas_call(
        _qkv_body,
        grid=(HQ // _HG,),
        in_specs=[
            pl.BlockSpec((SQ, D), lambda g: (0, 0)),
            pl.BlockSpec((D, _WG), lambda g: (0, g)),
            pl.BlockSpec((D, _WG), lambda g: (0, g)),
            pl.BlockSpec((D, _WG), lambda g: (0, g)),
        ],
        out_specs=[pl.BlockSpec((_HG, SQ, DH), lambda g: (g, 0, 0))] * 3,
        out_shape=[jax.ShapeDtypeStruct((HQ, SQ, DH), jnp.bfloat16)] * 3,
        scratch_shapes=[
            pltpu.VMEM((SQ, DH), jnp.float32),
            pltpu.VMEM((SQ, DH), jnp.float32),
        ],
        compiler_params=pltpu.CompilerParams(
            vmem_limit_bytes=60 * 1024 * 1024),
    )(x, Wq, Wk, Wv)


def _fused_body(q_ref, k_ref, v_ref, o_ref,
                ckL, cvL, ckR, cvR, cko, cvo, l_ref, ssem, rsem):
    my = lax.axis_index("i")
    left = lax.rem(my + N_DEV - 1, N_DEV)
    right = lax.rem(my + 1, N_DEV)

    barrier = pltpu.get_barrier_semaphore()
    for nbr in (left, right):
        pl.semaphore_signal(
            barrier, inc=1, device_id=(nbr,),
            device_id_type=pl.DeviceIdType.MESH,
        )
    pl.semaphore_wait(barrier, 2)

    H2 = HQ // 2

    def rc(src, dst, i, dev):
        return pltpu.make_async_remote_copy(
            src_ref=src, dst_ref=dst,
            send_sem=ssem.at[i], recv_sem=rsem.at[i],
            device_id=(dev,), device_id_type=pl.DeviceIdType.MESH,
        )

    def attn_chunk(kc, vc, first=False):
        def head_body(h, _):
            kh = kc[h]
            vh = vc[h]
            for qb in range(SQ // QB):
                qs = qb * QB
                qh = q_ref[h, pl.ds(qs, QB), :]
                s = lax.dot_general(
                    qh, kh, (((1,), (1,)), ((), ())),
                    preferred_element_type=jnp.float32,
                )
                p = jnp.exp2(s.astype(jnp.bfloat16))
                pv = jnp.dot(p, vh, preferred_element_type=jnp.float32)
                lsum = jnp.broadcast_to(
                    jnp.sum(p, axis=1, keepdims=True,
                            dtype=jnp.float32), (QB, DH))
                if first:
                    o_ref[h, pl.ds(qs, QB), :] = pv
                    l_ref[h, pl.ds(qs, QB), :] = lsum
                else:
                    o_ref[h, pl.ds(qs, QB), :] = (
                        o_ref[h, pl.ds(qs, QB), :] + pv)
                    l_ref[h, pl.ds(qs, QB), :] = (
                        l_ref[h, pl.ds(qs, QB), :] + lsum)
            return 0

        lax.fori_loop(0, HQ, head_body, 0)

    hA = pl.ds(0, H2)
    hB = pl.ds(H2, H2)
    r1 = rc(k_ref.at[hA], ckL.at[hA], 0, right)
    r2 = rc(v_ref.at[hA], cvL.at[hA], 1, right)
    r3 = rc(k_ref.at[hB], ckL.at[hB], 2, right)
    r4 = rc(v_ref.at[hB], cvL.at[hB], 3, right)
    l1 = rc(k_ref.at[hB], ckR.at[hB], 4, left)
    l2 = rc(v_ref.at[hB], cvR.at[hB], 5, left)
    l3 = rc(k_ref.at[hA], ckR.at[hA], 6, left)
    l4 = rc(v_ref.at[hA], cvR.at[hA], 7, left)
    for d in (r1, r2, r3, r4, l1, l2, l3, l4):
        d.start()

    attn_chunk(k_ref, v_ref, first=True)

    r1.wait()
    r2.wait()
    r5 = rc(ckL.at[hA], cko.at[hA], 8, right)
    r6 = rc(cvL.at[hA], cvo.at[hA], 9, right)
    r5.start()
    r6.start()
    l1.wait()
    l2.wait()
    l5 = rc(ckR.at[hB], cko.at[hB], 10, left)
    l6 = rc(cvR.at[hB], cvo.at[hB], 11, left)
    l5.start()
    l6.start()

    r3.wait()
    r4.wait()
    attn_chunk(ckL, cvL)
    l3.wait()
    l4.wait()
    attn_chunk(ckR, cvR)

    r5.wait()
    r6.wait()
    l5.wait()
    l6.wait()
    attn_chunk(cko, cvo)

    def norm_body(h, _):
        for qb in range(SQ // QB):
            qs = qb * QB
            o_ref[h, pl.ds(qs, QB), :] = (
                o_ref[h, pl.ds(qs, QB), :] / l_ref[h, pl.ds(qs, QB), :])
        return 0

    lax.fori_loop(0, HQ, norm_body, 0)


def _fused(q, k, v):
    return pl.pallas_call(
        _fused_body,
        in_specs=[pl.BlockSpec(memory_space=pltpu.MemorySpace.VMEM)] * 3,
        out_specs=pl.BlockSpec(memory_space=pltpu.MemorySpace.VMEM),
        out_shape=jax.ShapeDtypeStruct((HQ, SQ, DH), jnp.float32),
        scratch_shapes=[
            pltpu.VMEM((HQ, SQ, DH), jnp.bfloat16),
            pltpu.VMEM((HQ, SQ, DH), jnp.bfloat16),
            pltpu.VMEM((HQ, SQ, DH), jnp.bfloat16),
            pltpu.VMEM((HQ, SQ, DH), jnp.bfloat16),
            pltpu.VMEM((HQ, SQ, DH), jnp.bfloat16),
            pltpu.VMEM((HQ, SQ, DH), jnp.bfloat16),
            pltpu.VMEM((HQ, SQ, DH), jnp.float32),
            pltpu.SemaphoreType.DMA((12,)),
            pltpu.SemaphoreType.DMA((12,)),
        ],
        compiler_params=pltpu.CompilerParams(
            collective_id=0,
            vmem_limit_bytes=62 * 1024 * 1024,
        ),
    )(q, k, v)


def _proj_body(c_ref, w_ref, o_ref):
    h = pl.program_id(0)
    ctx = c_ref[0].astype(jnp.bfloat16)
    part = jnp.dot(ctx, w_ref[...].astype(jnp.bfloat16),
                   preferred_element_type=jnp.float32)

    @pl.when(h == 0)
    def _():
        o_ref[...] = part

    @pl.when(h > 0)
    def _():
        o_ref[...] += part


def _proj(ctx, Wo):
    return pl.pallas_call(
        _proj_body,
        grid=(HQ,),
        in_specs=[
            pl.BlockSpec((1, SQ, DH), lambda h: (h, 0, 0)),
            pl.BlockSpec((DH, D), lambda h: (h, 0)),
        ],
        out_specs=pl.BlockSpec((SQ, D), lambda h: (0, 0)),
        out_shape=jax.ShapeDtypeStruct((SQ, D), jnp.float32),
    )(ctx, Wo)


def kernel(x, Wq, Wk, Wv, Wo):
    x2 = x.reshape(SQ, D)
    q, k, v = _qkv(x2, Wq, Wk, Wv)
    ctx = _fused(q, k, v)
    out = _proj(ctx, Wo)
    return out.reshape(1, SQ, D)


# device time: 273609 ns/iter; 1.0333x vs baseline; 1.0333x over previous
import math

import jax
import jax.numpy as jnp
from jax import lax
from jax.experimental import pallas as pl
from jax.experimental.pallas import tpu as pltpu

N_DEV = 4
SQ = 2048
D = 1024
HQ = 8
DH = 128
QB = 512
SCALE = 0.08838834764831843 * 1.4426950408889634


def _rope_tables(off):
    ri = lax.broadcasted_iota(jnp.int32, (SQ, DH), 0).astype(jnp.float32)
    ci = lax.broadcasted_iota(jnp.int32, (SQ, DH), 1)
    f = (ci // 2).astype(jnp.float32)
    inv = jnp.exp(f * (-math.log(10000.0) / (DH // 2)))
    ang = (off.astype(jnp.float32) + ri) * inv
    return jnp.cos(ang), jnp.sin(ang)


def _rot_mat():
    i = lax.broadcasted_iota(jnp.int32, (DH, DH), 0)
    j = lax.broadcasted_iota(jnp.int32, (DH, DH), 1)
    plus = (j == i + 1) & (i % 2 == 0)
    minus = (j == i - 1) & (i % 2 == 1)
    return plus.astype(jnp.float32) - minus.astype(jnp.float32)


_HG = 4
_WG = _HG * DH


def _qkv_body(x_ref, wq_ref, wk_ref, wv_ref, q_ref, k_ref, v_ref,
              cos_ref, sin_ref):
    @pl.when(pl.program_id(0) == 0)
    def _():
        off = lax.axis_index("i") * SQ
        c, s = _rope_tables(off)
        cos_ref[...] = c
        sin_ref[...] = s

    cos = cos_ref[...]
    sin = sin_ref[...]
    rot = _rot_mat().astype(jnp.bfloat16)
    x = x_ref[...].astype(jnp.bfloat16)

    tq = jnp.dot(x, wq_ref[...].astype(jnp.bfloat16),
                 preferred_element_type=jnp.float32)
    tk = jnp.dot(x, wk_ref[...].astype(jnp.bfloat16),
                 preferred_element_type=jnp.float32)
    tv = jnp.dot(x, wv_ref[...].astype(jnp.bfloat16),
                 preferred_element_type=jnp.float32)
    for hh in range(_HG):
        t = tq[:, hh * DH:(hh + 1) * DH]
        tr = jnp.dot(t.astype(jnp.bfloat16), rot,
                     preferred_element_type=jnp.float32)
        q_ref[hh] = ((t * cos + tr * sin) * SCALE).astype(jnp.bfloat16)
        t = tk[:, hh * DH:(hh + 1) * DH]
        tr = jnp.dot(t.astype(jnp.bfloat16), rot,
                     preferred_element_type=jnp.float32)
        k_ref[hh] = (t * cos + tr * sin).astype(jnp.bfloat16)
        v_ref[hh] = tv[:, hh * DH:(hh + 1) * DH].astype(jnp.bfloat16)


def _qkv(x, Wq, Wk, Wv):
    return pl.pallas_call(
        _qkv_body,
        grid=(HQ // _HG,),
        in_specs=[
            pl.BlockSpec((SQ, D), lambda g: (0, 0)),
            pl.BlockSpec((D, _WG), lambda g: (0, g)),
            pl.BlockSpec((D, _WG), lambda g: (0, g)),
            pl.BlockSpec((D, _WG), lambda g: (0, g)),
        ],
        out_specs=[pl.BlockSpec((_HG, SQ, DH), lambda g: (g, 0, 0))] * 3,
        out_shape=[jax.ShapeDtypeStruct((HQ, SQ, DH), jnp.bfloat16)] * 3,
        scratch_shapes=[
            pltpu.VMEM((SQ, DH), jnp.float32),
            pltpu.VMEM((SQ, DH), jnp.float32),
        ],
        compiler_params=pltpu.CompilerParams(
            vmem_limit_bytes=60 * 1024 * 1024),
    )(x, Wq, Wk, Wv)


def _fused_body(q_ref, k_ref, v_ref, o_ref,
                ckL, cvL, ckR, cvR, cko, cvo, l_ref, ssem, rsem):
    my = lax.axis_index("i")
    left = lax.rem(my + N_DEV - 1, N_DEV)
    right = lax.rem(my + 1, N_DEV)

    barrier = pltpu.get_barrier_semaphore()
    for nbr in (left, right):
        pl.semaphore_signal(
            barrier, inc=1, device_id=(nbr,),
            device_id_type=pl.DeviceIdType.MESH,
        )
    pl.semaphore_wait(barrier, 2)

    H2 = HQ // 2

    def rc(src, dst, i, dev):
        return pltpu.make_async_remote_copy(
            src_ref=src, dst_ref=dst,
            send_sem=ssem.at[i], recv_sem=rsem.at[i],
            device_id=(dev,), device_id_type=pl.DeviceIdType.MESH,
        )

    def attn_chunk(kc, vc, first=False):
        def head_body(h, _):
            kh = kc[h]
            ve = jnp.concatenate(
                [vc[h], jnp.ones((SQ, DH), jnp.bfloat16)], axis=1)
            for qb in range(SQ // QB):
                qs = qb * QB
                qh = q_ref[h, pl.ds(qs, QB), :]
                s = lax.dot_general(
                    qh, kh, (((1,), (1,)), ((), ())),
                    preferred_element_type=jnp.float32,
                )
                p = jnp.exp2(s.astype(jnp.bfloat16))
                pv_l = jnp.dot(p, ve, preferred_element_type=jnp.float32)
                pv = pv_l[:, :DH]
                lsum = pv_l[:, DH:]
                if first:
                    o_ref[h, pl.ds(qs, QB), :] = pv
                    l_ref[h, pl.ds(qs, QB), :] = lsum
                else:
                    o_ref[h, pl.ds(qs, QB), :] = (
                        o_ref[h, pl.ds(qs, QB), :] + pv)
                    l_ref[h, pl.ds(qs, QB), :] = (
                        l_ref[h, pl.ds(qs, QB), :] + lsum)
            return 0

        lax.fori_loop(0, HQ, head_body, 0)

    hA = pl.ds(0, H2)
    hB = pl.ds(H2, H2)
    r1 = rc(k_ref.at[hA], ckL.at[hA], 0, right)
    r2 = rc(v_ref.at[hA], cvL.at[hA], 1, right)
    r3 = rc(k_ref.at[hB], ckL.at[hB], 2, right)
    r4 = rc(v_ref.at[hB], cvL.at[hB], 3, right)
    l1 = rc(k_ref.at[hB], ckR.at[hB], 4, left)
    l2 = rc(v_ref.at[hB], cvR.at[hB], 5, left)
    l3 = rc(k_ref.at[hA], ckR.at[hA], 6, left)
    l4 = rc(v_ref.at[hA], cvR.at[hA], 7, left)
    for d in (r1, r2, r3, r4, l1, l2, l3, l4):
        d.start()

    attn_chunk(k_ref, v_ref, first=True)

    r1.wait()
    r2.wait()
    r5 = rc(ckL.at[hA], cko.at[hA], 8, right)
    r6 = rc(cvL.at[hA], cvo.at[hA], 9, right)
    r5.start()
    r6.start()
    l1.wait()
    l2.wait()
    l5 = rc(ckR.at[hB], cko.at[hB], 10, left)
    l6 = rc(cvR.at[hB], cvo.at[hB], 11, left)
    l5.start()
    l6.start()

    r3.wait()
    r4.wait()
    attn_chunk(ckL, cvL)
    l3.wait()
    l4.wait()
    attn_chunk(ckR, cvR)

    r5.wait()
    r6.wait()
    l5.wait()
    l6.wait()
    attn_chunk(cko, cvo)

    def norm_body(h, _):
        for qb in range(SQ // QB):
            qs = qb * QB
            o_ref[h, pl.ds(qs, QB), :] = (
                o_ref[h, pl.ds(qs, QB), :] / l_ref[h, pl.ds(qs, QB), :])
        return 0

    lax.fori_loop(0, HQ, norm_body, 0)


def _fused(q, k, v):
    return pl.pallas_call(
        _fused_body,
        in_specs=[pl.BlockSpec(memory_space=pltpu.MemorySpace.VMEM)] * 3,
        out_specs=pl.BlockSpec(memory_space=pltpu.MemorySpace.VMEM),
        out_shape=jax.ShapeDtypeStruct((HQ, SQ, DH), jnp.float32),
        scratch_shapes=[
            pltpu.VMEM((HQ, SQ, DH), jnp.bfloat16),
            pltpu.VMEM((HQ, SQ, DH), jnp.bfloat16),
            pltpu.VMEM((HQ, SQ, DH), jnp.bfloat16),
            pltpu.VMEM((HQ, SQ, DH), jnp.bfloat16),
            pltpu.VMEM((HQ, SQ, DH), jnp.bfloat16),
            pltpu.VMEM((HQ, SQ, DH), jnp.bfloat16),
            pltpu.VMEM((HQ, SQ, DH), jnp.float32),
            pltpu.SemaphoreType.DMA((12,)),
            pltpu.SemaphoreType.DMA((12,)),
        ],
        compiler_params=pltpu.CompilerParams(
            collective_id=0,
            vmem_limit_bytes=62 * 1024 * 1024,
        ),
    )(q, k, v)


def _proj_body(c_ref, w_ref, o_ref):
    h = pl.program_id(0)
    ctx = c_ref[0].astype(jnp.bfloat16)
    part = jnp.dot(ctx, w_ref[...].astype(jnp.bfloat16),
                   preferred_element_type=jnp.float32)

    @pl.when(h == 0)
    def _():
        o_ref[...] = part

    @pl.when(h > 0)
    def _():
        o_ref[...] += part


def _proj(ctx, Wo):
    return pl.pallas_call(
        _proj_body,
        grid=(HQ,),
        in_specs=[
            pl.BlockSpec((1, SQ, DH), lambda h: (h, 0, 0)),
            pl.BlockSpec((DH, D), lambda h: (h, 0)),
        ],
        out_specs=pl.BlockSpec((SQ, D), lambda h: (0, 0)),
        out_shape=jax.ShapeDtypeStruct((SQ, D), jnp.float32),
    )(ctx, Wo)


def kernel(x, Wq, Wk, Wv, Wo):
    x2 = x.reshape(SQ, D)
    q, k, v = _qkv(x2, Wq, Wk, Wv)
    ctx = _fused(q, k, v)
    out = _proj(ctx, Wo)
    return out.reshape(1, SQ, D)
